# Initial kernel scaffold; baseline (speedup 1.0000x reference)
import functools

import jax
import jax.numpy as jnp
from jax import lax
from jax.experimental import pallas as pl
from jax.experimental.pallas import tpu as pltpu
from jax.experimental.pallas import tpu_sc as plsc

N = 10000
NP = 10240
E = 160000
NFEAT = 128
NHID = 64

NC = 2
NS = 16
NW = NC * NS
EPW = 5120
EP = NW * EPW
CH = 128
NCHUNK = EPW // CH
NPT = NP // NS

_MESH = dict(core_axis_name="c", subcore_axis_name="s", num_cores=NC,
             num_subcores=NS)


def _degree_kernel(src_flat, dst_flat):
    mesh = plsc.VectorSubcoreMesh(**_MESH)

    @functools.partial(
        pl.kernel,
        out_type=[jax.ShapeDtypeStruct((NW, NP), jnp.float32),
                  jax.ShapeDtypeStruct((NW, NP), jnp.float32)],
        mesh=mesh,
        scratch_types=[pltpu.VMEM((EPW,), jnp.int32),
                       pltpu.VMEM((EPW,), jnp.int32),
                       pltpu.VMEM((NP,), jnp.float32),
                       pltpu.VMEM((NP,), jnp.float32)],
    )
    def k(src_hbm, dst_hbm, outd_hbm, ind_hbm, src_v, dst_v, hs_v, hd_v):
        cid = lax.axis_index("c")
        sid = lax.axis_index("s")
        wid = sid * NC + cid
        pltpu.sync_copy(src_hbm.at[wid], src_v)
        pltpu.sync_copy(dst_hbm.at[wid], dst_v)

        zeros16 = jnp.zeros((16,), jnp.float32)

        def zbody(i, carry):
            hs_v[pl.ds(i * 16, 16)] = zeros16
            hd_v[pl.ds(i * 16, 16)] = zeros16
            return carry
        lax.fori_loop(0, NP // 16, zbody, 0)

        ones16 = jnp.ones((16,), jnp.float32)

        def body(i, carry):
            si = src_v[pl.ds(i * 16, 16)]
            plsc.addupdate_scatter(hs_v, [si], ones16)
            di = dst_v[pl.ds(i * 16, 16)]
            plsc.addupdate_scatter(hd_v, [di], ones16)
            return carry
        lax.fori_loop(0, EPW // 16, body, 0)

        pltpu.sync_copy(hs_v, outd_hbm.at[wid])
        pltpu.sync_copy(hd_v, ind_hbm.at[wid])

    return k(src_flat, dst_flat)


def _mp_kernel(feat, src3, dst3, d):
    mesh = plsc.VectorSubcoreMesh(**_MESH)

    @functools.partial(
        pl.kernel,
        out_type=jax.ShapeDtypeStruct((NC, NP, d), jnp.float32),
        mesh=mesh,
        scratch_types=[pltpu.VMEM((NCHUNK, CH), jnp.int32),
                       pltpu.VMEM((NCHUNK, CH), jnp.int32),
                       pltpu.VMEM((CH, d), jnp.float32),
                       pltpu.VMEM((CH, d), jnp.float32),
                       pltpu.VMEM_SHARED((NP, d), jnp.float32),
                       pltpu.SemaphoreType.DMA],
    )
    def k(feat_hbm, src_hbm, dst_hbm, out_hbm, src_v, dst_v, rows_a, rows_b,
          acc_sh, gsem):
        cid = lax.axis_index("c")
        sid = lax.axis_index("s")
        wid = sid * NC + cid
        pltpu.sync_copy(src_hbm.at[wid], src_v)
        pltpu.sync_copy(dst_hbm.at[wid], dst_v)

        zeros16 = jnp.zeros((16,), jnp.float32)

        def zbody(i, carry):
            for q in range(d // 16):
                rows_b[i, pl.ds(q * 16, 16)] = zeros16
            return carry
        lax.fori_loop(0, CH, zbody, 0)
        for t in range(NPT // CH):
            pltpu.sync_copy(rows_b, acc_sh.at[pl.ds(sid * NPT + t * CH, CH)])
        plsc.subcore_barrier()

        def chunk(j, carry):
            cp = pltpu.async_copy(feat_hbm.at[src_v.at[j]], rows_a, gsem)
            cp.wait()
            pltpu.sync_copy(rows_a, acc_sh.at[dst_v.at[j]], add=True)
            return carry
        lax.fori_loop(0, NCHUNK, chunk, 0)

        plsc.subcore_barrier()
        pltpu.sync_copy(acc_sh.at[pl.ds(sid * NPT, NPT)],
                        out_hbm.at[cid, pl.ds(sid * NPT, NPT)])

    return k(feat, src3, dst3)


_BM = 512


def _norm_feat_kernel(outd_part, ind_part, x):
    grid = NP // _BM

    def body(od_ref, id_ref, x_ref, on_ref, in_ref, f_ref):
        i = pl.program_id(0)
        od = jnp.sum(od_ref[...], axis=0)
        idg = jnp.sum(id_ref[...], axis=0)
        onorm = lax.rsqrt(jnp.maximum(od, 1.0))[:, None]
        inorm = lax.rsqrt(jnp.maximum(idg, 1.0))[:, None]
        on_ref[...] = onorm
        in_ref[...] = inorm
        rows = i * _BM + lax.broadcasted_iota(jnp.int32, (_BM, 1), 0)
        f_ref[...] = jnp.where(rows < N, x_ref[...] * onorm, 0.0)

    return pl.pallas_call(
        body,
        grid=(grid,),
        in_specs=[pl.BlockSpec((NW, _BM), lambda i: (0, i)),
                  pl.BlockSpec((NW, _BM), lambda i: (0, i)),
                  pl.BlockSpec((_BM, NFEAT), lambda i: (i, 0))],
        out_specs=[pl.BlockSpec((_BM, 1), lambda i: (i, 0)),
                   pl.BlockSpec((_BM, 1), lambda i: (i, 0)),
                   pl.BlockSpec((_BM, NFEAT), lambda i: (i, 0))],
        out_shape=[jax.ShapeDtypeStruct((NP, 1), jnp.float32),
                   jax.ShapeDtypeStruct((NP, 1), jnp.float32),
                   jax.ShapeDtypeStruct((NP, NFEAT), jnp.float32)],
    )(outd_part, ind_part, x)


def _layer_kernel(aggp, inn, outn, Ws, bs, scale_out):
    din = aggp.shape[-1]
    grid = NP // _BM
    nout = len(Ws)

    def body(*refs):
        agg_ref, inn_ref, outn_ref = refs[:3]
        w_refs = refs[3:3 + nout]
        b_refs = refs[3 + nout:3 + 2 * nout]
        o_refs = refs[3 + 2 * nout:]
        agg = (agg_ref[0] + agg_ref[1]) * inn_ref[...]
        for w_ref, b_ref, o_ref, sc in zip(w_refs, b_refs, o_refs, scale_out):
            y = jnp.dot(agg, w_ref[...], preferred_element_type=jnp.float32)
            y = jnp.maximum(y + b_ref[...], 0.0)
            if sc:
                y = y * outn_ref[...]
            o_ref[...] = y

    in_specs = [pl.BlockSpec((NC, _BM, din), lambda i: (0, i, 0)),
                pl.BlockSpec((_BM, 1), lambda i: (i, 0)),
                pl.BlockSpec((_BM, 1), lambda i: (i, 0))]
    for W in Ws:
        in_specs.append(pl.BlockSpec(W.shape, lambda i: (0, 0)))
    bs2 = [b[None, :] for b in bs]
    for b2 in bs2:
        in_specs.append(pl.BlockSpec(b2.shape, lambda i: (0, 0)))

    out = pl.pallas_call(
        body,
        grid=(grid,),
        in_specs=in_specs,
        out_specs=[pl.BlockSpec((_BM, W.shape[1]), lambda i: (i, 0))
                   for W in Ws],
        out_shape=[jax.ShapeDtypeStruct((NP, W.shape[1]), jnp.float32)
                   for W in Ws],
    )(aggp, inn, outn, *Ws, *bs2)
    return out


def _xhat_kernel(aggp, inn, W, b):
    grid = NP // _BM
    din = aggp.shape[-1]

    def body(agg_ref, inn_ref, w_ref, b_ref, o_ref):
        agg = (agg_ref[0] + agg_ref[1]) * inn_ref[...]
        y = jnp.dot(agg, w_ref[...], preferred_element_type=jnp.float32)
        o_ref[...] = jnp.maximum(y + b_ref[...], 0.0)

    return pl.pallas_call(
        body,
        grid=(grid,),
        in_specs=[pl.BlockSpec((NC, _BM, din), lambda i: (0, i, 0)),
                  pl.BlockSpec((_BM, 1), lambda i: (i, 0)),
                  pl.BlockSpec(W.shape, lambda i: (0, 0)),
                  pl.BlockSpec((1, W.shape[1]), lambda i: (0, 0))],
        out_specs=pl.BlockSpec((_BM, W.shape[1]), lambda i: (i, 0)),
        out_shape=jax.ShapeDtypeStruct((N, W.shape[1]), jnp.float32),
    )(aggp, inn, W, b[None, :])


_BMI = 256
_BNJ = 2048


def _struct_kernel(s):
    gj = NP // _BNJ
    gi = pl.cdiv(N, _BMI)

    def body(si_ref, sj_ref, o_ref):
        o_ref[...] = lax.dot_general(
            si_ref[...], sj_ref[...], (((1,), (1,)), ((), ())),
            preferred_element_type=jnp.float32)

    return pl.pallas_call(
        body,
        grid=(gj, gi),
        in_specs=[pl.BlockSpec((_BMI, NHID), lambda j, i: (i, 0)),
                  pl.BlockSpec((_BNJ, NHID), lambda j, i: (j, 0))],
        out_specs=pl.BlockSpec((_BMI, _BNJ), lambda j, i: (i, j)),
        out_shape=jax.ShapeDtypeStruct((N, N), jnp.float32),
    )(s, s)


def kernel(x, edge_index, W_e1, b_e1, W_e2, b_e2, W_a1, b_a1, W_a2, b_a2,
           W_s1, b_s1):
    src = edge_index[0].astype(jnp.int32)
    dst = edge_index[1].astype(jnp.int32)
    pad = jnp.full((EP - E,), N, jnp.int32)
    src_p = jnp.concatenate([src, pad]).reshape(NW, EPW)
    dst_p = jnp.concatenate([dst, pad]).reshape(NW, EPW)
    src3 = src_p.reshape(NW, NCHUNK, CH)
    dst3 = dst_p.reshape(NW, NCHUNK, CH)

    outd_part, ind_part = _degree_kernel(src_p, dst_p)
    outn, inn, feat1 = _norm_feat_kernel(outd_part, ind_part, x)

    agg1 = _mp_kernel(feat1, src3, dst3, NFEAT)
    (feat2,) = _layer_kernel(agg1, inn, outn, [W_e1], [b_e1], [True])

    agg2 = _mp_kernel(feat2, src3, dst3, NHID)
    (feat3,) = _layer_kernel(agg2, inn, outn, [W_e2], [b_e2], [True])

    agg3 = _mp_kernel(feat3, src3, dst3, NHID)
    feat4, s = _layer_kernel(agg3, inn, outn, [W_a1, W_s1], [b_a1, b_s1],
                             [True, False])

    agg4 = _mp_kernel(feat4, src3, dst3, NHID)
    x_hat = _xhat_kernel(agg4, inn, W_a2, b_a2)

    struct = _struct_kernel(s)
    return (struct, x_hat)


# speedup vs baseline: 3.1686x; 3.1686x over previous
import functools

import jax
import jax.numpy as jnp
from jax import lax
from jax.experimental import pallas as pl
from jax.experimental.pallas import tpu as pltpu
from jax.experimental.pallas import tpu_sc as plsc

N = 10000
NP = 10240
E = 160000
NFEAT = 128
NHID = 64

NC = 2
NS = 16
NW = NC * NS
EPW = 5120
EP = NW * EPW
CH = 128
NCHUNK = EPW // CH
NPT = NP // NS

_MESH = dict(core_axis_name="c", subcore_axis_name="s", num_cores=NC,
             num_subcores=NS)


def _degree_kernel(src_flat, dst_flat):
    mesh = plsc.VectorSubcoreMesh(**_MESH)

    @functools.partial(
        pl.kernel,
        out_type=[jax.ShapeDtypeStruct((NW, NP), jnp.float32),
                  jax.ShapeDtypeStruct((NW, NP), jnp.float32)],
        mesh=mesh,
        scratch_types=[pltpu.VMEM((EPW,), jnp.int32),
                       pltpu.VMEM((EPW,), jnp.int32),
                       pltpu.VMEM((NP,), jnp.float32),
                       pltpu.VMEM((NP,), jnp.float32)],
        compiler_params=pltpu.CompilerParams(needs_layout_passes=False),
    )
    def k(src_hbm, dst_hbm, outd_hbm, ind_hbm, src_v, dst_v, hs_v, hd_v):
        cid = lax.axis_index("c")
        sid = lax.axis_index("s")
        wid = sid * NC + cid
        pltpu.sync_copy(src_hbm.at[wid], src_v)
        pltpu.sync_copy(dst_hbm.at[wid], dst_v)

        zeros16 = jnp.zeros((16,), jnp.float32)

        def zbody(i, carry):
            hs_v[pl.ds(i * 16, 16)] = zeros16
            hd_v[pl.ds(i * 16, 16)] = zeros16
            return carry
        lax.fori_loop(0, NP // 16, zbody, 0)

        ones16 = jnp.ones((16,), jnp.float32)

        def body(i, carry):
            si = src_v[pl.ds(i * 16, 16)]
            plsc.addupdate_scatter(hs_v, [si], ones16)
            di = dst_v[pl.ds(i * 16, 16)]
            plsc.addupdate_scatter(hd_v, [di], ones16)
            return carry
        lax.fori_loop(0, EPW // 16, body, 0)

        pltpu.sync_copy(hs_v, outd_hbm.at[wid])
        pltpu.sync_copy(hd_v, ind_hbm.at[wid])

    return k(src_flat, dst_flat)


def _mp_kernel(feat, src3, dst3, d):
    mesh = plsc.VectorSubcoreMesh(**_MESH)

    @functools.partial(
        pl.kernel,
        out_type=jax.ShapeDtypeStruct((NC, NP, d), jnp.float32),
        mesh=mesh,
        scratch_types=[pltpu.VMEM((NCHUNK, CH), jnp.int32),
                       pltpu.VMEM((NCHUNK, CH), jnp.int32),
                       pltpu.VMEM((CH, d), jnp.float32),
                       pltpu.VMEM((CH, d), jnp.float32),
                       pltpu.VMEM_SHARED((NP, d), jnp.float32),
                       pltpu.SemaphoreType.DMA],
        compiler_params=pltpu.CompilerParams(use_tc_tiling_on_sc=False),
    )
    def k(feat_hbm, src_hbm, dst_hbm, out_hbm, src_v, dst_v, rows_a, rows_b,
          acc_sh, gsem):
        cid = lax.axis_index("c")
        sid = lax.axis_index("s")
        wid = sid * NC + cid
        pltpu.sync_copy(src_hbm.at[wid], src_v)
        pltpu.sync_copy(dst_hbm.at[wid], dst_v)

        zeros16 = jnp.zeros((16,), jnp.float32)

        def zbody(i, carry):
            for q in range(d // 16):
                rows_b[i, pl.ds(q * 16, 16)] = zeros16
            return carry
        lax.fori_loop(0, CH, zbody, 0)
        for t in range(NPT // CH):
            pltpu.sync_copy(rows_b, acc_sh.at[pl.ds(sid * NPT + t * CH, CH)])
        plsc.subcore_barrier()

        def chunk(j, carry):
            cp = pltpu.async_copy(feat_hbm.at[src_v.at[j]], rows_a, gsem)
            cp.wait()
            pltpu.sync_copy(rows_a, acc_sh.at[dst_v.at[j]], add=True)
            return carry
        lax.fori_loop(0, NCHUNK, chunk, 0)

        plsc.subcore_barrier()
        pltpu.sync_copy(acc_sh.at[pl.ds(sid * NPT, NPT)],
                        out_hbm.at[cid, pl.ds(sid * NPT, NPT)])

    return k(feat, src3, dst3)


_BM = 512


def _norm_feat_kernel(outd_part, ind_part, x):
    grid = NP // _BM

    def body(od_ref, id_ref, x_ref, on_ref, in_ref, f_ref):
        i = pl.program_id(0)
        od = jnp.sum(od_ref[...], axis=0)
        idg = jnp.sum(id_ref[...], axis=0)
        onorm = lax.rsqrt(jnp.maximum(od, 1.0))[:, None]
        inorm = lax.rsqrt(jnp.maximum(idg, 1.0))[:, None]
        on_ref[...] = onorm
        in_ref[...] = inorm
        rows = i * _BM + lax.broadcasted_iota(jnp.int32, (_BM, 1), 0)
        f_ref[...] = jnp.where(rows < N, x_ref[...] * onorm, 0.0)

    return pl.pallas_call(
        body,
        grid=(grid,),
        in_specs=[pl.BlockSpec((NW, _BM), lambda i: (0, i)),
                  pl.BlockSpec((NW, _BM), lambda i: (0, i)),
                  pl.BlockSpec((_BM, NFEAT), lambda i: (i, 0))],
        out_specs=[pl.BlockSpec((_BM, 1), lambda i: (i, 0)),
                   pl.BlockSpec((_BM, 1), lambda i: (i, 0)),
                   pl.BlockSpec((_BM, NFEAT), lambda i: (i, 0))],
        out_shape=[jax.ShapeDtypeStruct((NP, 1), jnp.float32),
                   jax.ShapeDtypeStruct((NP, 1), jnp.float32),
                   jax.ShapeDtypeStruct((NP, NFEAT), jnp.float32)],
    )(outd_part, ind_part, x)


def _layer_kernel(aggp, inn, outn, Ws, bs, scale_out):
    din = aggp.shape[-1]
    grid = NP // _BM
    nout = len(Ws)

    def body(*refs):
        agg_ref, inn_ref, outn_ref = refs[:3]
        w_refs = refs[3:3 + nout]
        b_refs = refs[3 + nout:3 + 2 * nout]
        o_refs = refs[3 + 2 * nout:]
        agg = (agg_ref[0] + agg_ref[1]) * inn_ref[...]
        for w_ref, b_ref, o_ref, sc in zip(w_refs, b_refs, o_refs, scale_out):
            y = jnp.dot(agg, w_ref[...], preferred_element_type=jnp.float32)
            y = jnp.maximum(y + b_ref[...], 0.0)
            if sc:
                y = y * outn_ref[...]
            o_ref[...] = y

    in_specs = [pl.BlockSpec((NC, _BM, din), lambda i: (0, i, 0)),
                pl.BlockSpec((_BM, 1), lambda i: (i, 0)),
                pl.BlockSpec((_BM, 1), lambda i: (i, 0))]
    for W in Ws:
        in_specs.append(pl.BlockSpec(W.shape, lambda i: (0, 0)))
    bs2 = [b[None, :] for b in bs]
    for b2 in bs2:
        in_specs.append(pl.BlockSpec(b2.shape, lambda i: (0, 0)))

    out = pl.pallas_call(
        body,
        grid=(grid,),
        in_specs=in_specs,
        out_specs=[pl.BlockSpec((_BM, W.shape[1]), lambda i: (i, 0))
                   for W in Ws],
        out_shape=[jax.ShapeDtypeStruct((NP, W.shape[1]), jnp.float32)
                   for W in Ws],
    )(aggp, inn, outn, *Ws, *bs2)
    return out


def _xhat_kernel(aggp, inn, W, b):
    grid = NP // _BM
    din = aggp.shape[-1]

    def body(agg_ref, inn_ref, w_ref, b_ref, o_ref):
        agg = (agg_ref[0] + agg_ref[1]) * inn_ref[...]
        y = jnp.dot(agg, w_ref[...], preferred_element_type=jnp.float32)
        o_ref[...] = jnp.maximum(y + b_ref[...], 0.0)

    return pl.pallas_call(
        body,
        grid=(grid,),
        in_specs=[pl.BlockSpec((NC, _BM, din), lambda i: (0, i, 0)),
                  pl.BlockSpec((_BM, 1), lambda i: (i, 0)),
                  pl.BlockSpec(W.shape, lambda i: (0, 0)),
                  pl.BlockSpec((1, W.shape[1]), lambda i: (0, 0))],
        out_specs=pl.BlockSpec((_BM, W.shape[1]), lambda i: (i, 0)),
        out_shape=jax.ShapeDtypeStruct((N, W.shape[1]), jnp.float32),
    )(aggp, inn, W, b[None, :])


_BMI = 256
_BNJ = 2048


def _struct_kernel(s):
    gj = NP // _BNJ
    gi = pl.cdiv(N, _BMI)

    def body(si_ref, sj_ref, o_ref):
        o_ref[...] = lax.dot_general(
            si_ref[...], sj_ref[...], (((1,), (1,)), ((), ())),
            preferred_element_type=jnp.float32)

    return pl.pallas_call(
        body,
        grid=(gj, gi),
        in_specs=[pl.BlockSpec((_BMI, NHID), lambda j, i: (i, 0)),
                  pl.BlockSpec((_BNJ, NHID), lambda j, i: (j, 0))],
        out_specs=pl.BlockSpec((_BMI, _BNJ), lambda j, i: (i, j)),
        out_shape=jax.ShapeDtypeStruct((N, N), jnp.float32),
    )(s, s)


def kernel(x, edge_index, W_e1, b_e1, W_e2, b_e2, W_a1, b_a1, W_a2, b_a2,
           W_s1, b_s1):
    src = edge_index[0].astype(jnp.int32)
    dst = edge_index[1].astype(jnp.int32)
    pad = jnp.full((EP - E,), N, jnp.int32)
    src_p = jnp.concatenate([src, pad]).reshape(NW, EPW)
    dst_p = jnp.concatenate([dst, pad]).reshape(NW, EPW)
    src3 = src_p.reshape(NW, NCHUNK, CH)
    dst3 = dst_p.reshape(NW, NCHUNK, CH)

    outd_part, ind_part = _degree_kernel(src_p, dst_p)
    outn, inn, feat1 = _norm_feat_kernel(outd_part, ind_part, x)

    agg1 = _mp_kernel(feat1, src3, dst3, NFEAT)
    (feat2,) = _layer_kernel(agg1, inn, outn, [W_e1], [b_e1], [True])

    agg2 = _mp_kernel(feat2, src3, dst3, NHID)
    (feat3,) = _layer_kernel(agg2, inn, outn, [W_e2], [b_e2], [True])

    agg3 = _mp_kernel(feat3, src3, dst3, NHID)
    feat4, s = _layer_kernel(agg3, inn, outn, [W_a1, W_s1], [b_a1, b_s1],
                             [True, False])

    agg4 = _mp_kernel(feat4, src3, dst3, NHID)
    x_hat = _xhat_kernel(agg4, inn, W_a2, b_a2)

    struct = _struct_kernel(s)
    return (struct, x_hat)
